# baseline (speedup 1.0000x reference)
import functools

import jax
import jax.numpy as jnp
from jax import lax
from jax.experimental import pallas as pl
from jax.experimental.pallas import tpu as pltpu
from jax.experimental.pallas import tpu_sc as plsc

D_FEAT = 64
D_MODEL = 128
CHUNK = 64
NC, NS = 2, 16
NW = NC * NS
VBLK = 1024


def _tc_project_vocab(table, W, b2d):
    v = table.shape[0]
    grid = (v + VBLK - 1) // VBLK

    def body(t_ref, w_ref, b_ref, p_ref, s_ref):
        t = t_ref[...]
        y = jnp.dot(t, w_ref[...], preferred_element_type=jnp.float32) + b_ref[...]
        p_ref[...] = jax.nn.gelu(y)
        s_ref[...] = jnp.sum(t, axis=1)

    return pl.pallas_call(
        body,
        grid=(grid,),
        in_specs=[
            pl.BlockSpec((VBLK, D_FEAT), lambda i: (i, 0)),
            pl.BlockSpec((D_FEAT, D_MODEL), lambda i: (0, 0)),
            pl.BlockSpec((1, D_MODEL), lambda i: (0, 0)),
        ],
        out_specs=[
            pl.BlockSpec((VBLK, D_MODEL), lambda i: (i, 0)),
            pl.BlockSpec((VBLK,), lambda i: (i,)),
        ],
        out_shape=[
            jax.ShapeDtypeStruct((v, D_MODEL), jnp.float32),
            jax.ShapeDtypeStruct((v,), jnp.float32),
        ],
    )(table, W, b2d)


def _sc_gather_mask(idx3d, proj, rs):
    per_w = idx3d.shape[1]
    n = NW * per_w * CHUNK
    flat_w = per_w * CHUNK
    cg = CHUNK // 16
    nb = 2
    mesh = plsc.VectorSubcoreMesh(core_axis_name="c", subcore_axis_name="s")

    @functools.partial(
        pl.kernel,
        out_type=(
            jax.ShapeDtypeStruct((n, D_MODEL), jnp.float32),
            jax.ShapeDtypeStruct((n,), jnp.float32),
        ),
        mesh=mesh,
        scratch_types=[
            pltpu.VMEM(rs.shape, jnp.float32),
            pltpu.VMEM((per_w, CHUNK), jnp.int32),
            pltpu.VMEM((nb, CHUNK), jnp.float32),
            pltpu.VMEM((nb, CHUNK, D_MODEL), jnp.float32),
            pltpu.SemaphoreType.DMA((nb,)),
            pltpu.SemaphoreType.DMA((nb,)),
            pltpu.SemaphoreType.DMA((nb,)),
            pltpu.SemaphoreType.DMA,
        ],
        compiler_params=pltpu.CompilerParams(needs_layout_passes=False),
    )
    def k(idx_hbm, proj_hbm, rs_hbm, out_hbm, mask_hbm,
          rs_v, idx_v, m_b, bufs, gsem, wsem, msem, rssem):
        wid = lax.axis_index("s") * NC + lax.axis_index("c")
        base = wid * per_w
        pltpu.sync_copy(idx_hbm.at[wid], idx_v)
        rs_cp = pltpu.async_copy(rs_hbm, rs_v, rssem)
        for j in range(nb - 1):
            pltpu.async_copy(proj_hbm.at[idx_v.at[j]], bufs.at[j], gsem.at[j])
        rs_cp.wait()

        def body(j, carry):
            p = lax.rem(j, nb)
            pltpu.make_async_copy(proj_hbm.at[idx_v.at[j]], bufs.at[p],
                                  gsem.at[p]).wait()
            pltpu.async_copy(bufs.at[p],
                             out_hbm.at[pl.ds((base + j) * CHUNK, CHUNK)],
                             wsem.at[p])
            nxt = j + nb - 1
            q = lax.rem(nxt, nb)

            @pl.when(nxt < per_w)
            def _():
                @pl.when(j >= 1)
                def _():
                    pltpu.make_async_copy(
                        bufs.at[q], out_hbm.at[pl.ds(base * CHUNK, CHUNK)],
                        wsem.at[q]).wait()

                pltpu.async_copy(proj_hbm.at[idx_v.at[nxt]], bufs.at[q],
                                 gsem.at[q])

            @pl.when(j >= nb)
            def _():
                pltpu.make_async_copy(
                    m_b.at[p], mask_hbm.at[pl.ds(base * CHUNK, CHUNK)],
                    msem.at[p]).wait()

            for c in range(cg):
                vidx = idx_v[j, pl.ds(c * 16, 16)]
                vals = plsc.load_gather(rs_v, [vidx])
                m_b[p, pl.ds(c * 16, 16)] = jnp.where(
                    vals == 0.0, 1.0, 0.0).astype(jnp.float32)
            pltpu.async_copy(m_b.at[p],
                             mask_hbm.at[pl.ds((base + j) * CHUNK, CHUNK)],
                             msem.at[p])
            return carry

        lax.fori_loop(0, per_w, body, 0)
        for p in range(nb):
            pltpu.make_async_copy(
                bufs.at[p], out_hbm.at[pl.ds(base * CHUNK, CHUNK)],
                wsem.at[p]).wait()
            pltpu.make_async_copy(
                m_b.at[p], mask_hbm.at[pl.ds(base * CHUNK, CHUNK)],
                msem.at[p]).wait()

    return k(idx3d, proj, rs)


def kernel(indices, table, W, b):
    bsz, seq = indices.shape
    n = bsz * seq
    idx3d = indices.reshape(NW, n // (NW * CHUNK), CHUNK).astype(jnp.int32)
    proj, rs = _tc_project_vocab(table, W, b.reshape(1, D_MODEL))
    out_flat, mask_flat = _sc_gather_mask(idx3d, proj, rs)
    out = out_flat.reshape(bsz, seq, D_MODEL)
    mask = mask_flat.reshape(bsz, seq).astype(bool)[:, None, None, :]
    return out, mask


# speedup vs baseline: 1.2717x; 1.2717x over previous
import functools

import jax
import jax.numpy as jnp
from jax import lax
from jax.experimental import pallas as pl
from jax.experimental.pallas import tpu as pltpu
from jax.experimental.pallas import tpu_sc as plsc

D_FEAT = 64
D_MODEL = 128
CHUNK = 128
NC, NS = 2, 16
NW = NC * NS
VBLK = 1024


def _tc_project_vocab(table, W, b2d):
    v = table.shape[0]
    grid = (v + VBLK - 1) // VBLK
    mb = VBLK // 128

    def body(t_ref, w_ref, b_ref, p_ref, s_ref):
        t = t_ref[...]
        y = jnp.dot(t, w_ref[...], preferred_element_type=jnp.float32) + b_ref[...]
        p_ref[...] = jax.nn.gelu(y)
        s_ref[...] = jnp.sum(t.reshape(mb, 128, D_FEAT), axis=-1)

    return pl.pallas_call(
        body,
        grid=(grid,),
        in_specs=[
            pl.BlockSpec((VBLK, D_FEAT), lambda i: (i, 0)),
            pl.BlockSpec((D_FEAT, D_MODEL), lambda i: (0, 0)),
            pl.BlockSpec((1, D_MODEL), lambda i: (0, 0)),
        ],
        out_specs=[
            pl.BlockSpec((VBLK, D_MODEL), lambda i: (i, 0)),
            pl.BlockSpec((mb, 128), lambda i: (i, 0)),
        ],
        out_shape=[
            jax.ShapeDtypeStruct((v, D_MODEL), jnp.float32),
            jax.ShapeDtypeStruct((grid * mb, 128), jnp.float32),
        ],
    )(table, W, b2d)


def _sc_gather(idx3d, proj):
    per_w = idx3d.shape[1]
    n = NW * per_w * CHUNK
    nb = 4
    mesh = plsc.VectorSubcoreMesh(core_axis_name="c", subcore_axis_name="s")

    @functools.partial(
        pl.kernel,
        out_type=jax.ShapeDtypeStruct((n, D_MODEL), jnp.float32),
        mesh=mesh,
        scratch_types=[
            pltpu.VMEM((per_w, CHUNK), jnp.int32),
            pltpu.VMEM((nb, CHUNK, D_MODEL), jnp.float32),
            pltpu.SemaphoreType.DMA((nb,)),
            pltpu.SemaphoreType.DMA((nb,)),
        ],
    )
    def k(idx_hbm, proj_hbm, out_hbm, idx_v, bufs, gsem, wsem):
        wid = lax.axis_index("s") * NC + lax.axis_index("c")
        base = wid * per_w
        pltpu.sync_copy(idx_hbm.at[wid], idx_v)

        for j in range(nb - 1):
            pltpu.async_copy(proj_hbm.at[idx_v.at[j]], bufs.at[j], gsem.at[j])

        def body(j, carry):
            p = lax.rem(j, nb)
            pltpu.make_async_copy(proj_hbm.at[idx_v.at[j]], bufs.at[p],
                                  gsem.at[p]).wait()
            pltpu.async_copy(bufs.at[p],
                             out_hbm.at[pl.ds((base + j) * CHUNK, CHUNK)],
                             wsem.at[p])
            nxt = j + nb - 1
            q = lax.rem(nxt, nb)

            @pl.when(nxt < per_w)
            def _():
                @pl.when(j >= 1)
                def _():
                    pltpu.make_async_copy(
                        bufs.at[q], out_hbm.at[pl.ds(base * CHUNK, CHUNK)],
                        wsem.at[q]).wait()

                pltpu.async_copy(proj_hbm.at[idx_v.at[nxt]], bufs.at[q],
                                 gsem.at[q])

            return carry

        lax.fori_loop(0, per_w, body, 0)
        for p in range(nb):
            pltpu.make_async_copy(
                bufs.at[p], out_hbm.at[pl.ds(base * CHUNK, CHUNK)],
                wsem.at[p]).wait()

    return k(idx3d, proj)


def _sc_mask(idx_flat, rs_flat):
    n = idx_flat.shape[0]
    per_w = n // NW
    groups = per_w // 16
    mesh = plsc.VectorSubcoreMesh(core_axis_name="c", subcore_axis_name="s")

    @functools.partial(
        pl.kernel,
        out_type=jax.ShapeDtypeStruct((n,), jnp.float32),
        mesh=mesh,
        scratch_types=[
            pltpu.VMEM(rs_flat.shape, jnp.float32),
            pltpu.VMEM((per_w,), jnp.int32),
            pltpu.VMEM((per_w,), jnp.float32),
        ],
        compiler_params=pltpu.CompilerParams(needs_layout_passes=False),
    )
    def k(idx_hbm, rs_hbm, out_hbm, rs_v, idx_v, m_v):
        wid = lax.axis_index("s") * NC + lax.axis_index("c")
        pltpu.sync_copy(rs_hbm, rs_v)
        pltpu.sync_copy(idx_hbm.at[pl.ds(wid * per_w, per_w)], idx_v)

        def body(j, carry):
            vidx = idx_v[pl.ds(j * 16, 16)]
            vals = plsc.load_gather(rs_v, [vidx])
            m_v[pl.ds(j * 16, 16)] = jnp.where(vals == 0.0, 1.0, 0.0).astype(jnp.float32)
            return carry

        lax.fori_loop(0, groups, body, 0)
        pltpu.sync_copy(m_v, out_hbm.at[pl.ds(wid * per_w, per_w)])

    return k(idx_flat, rs_flat)


def kernel(indices, table, W, b):
    bsz, seq = indices.shape
    n = bsz * seq
    idx3d = indices.reshape(NW, n // (NW * CHUNK), CHUNK).astype(jnp.int32)
    proj, rowsum = _tc_project_vocab(table, W, b.reshape(1, D_MODEL))
    out_flat = _sc_gather(idx3d, proj)
    mask_flat = _sc_mask(idx3d.reshape(n), rowsum.reshape(-1))
    out = out_flat.reshape(bsz, seq, D_MODEL)
    mask = mask_flat.reshape(bsz, seq).astype(bool)[:, None, None, :]
    return out, mask
